# Initial kernel scaffold; baseline (speedup 1.0000x reference)
import jax
import jax.numpy as jnp
from jax.experimental import pallas as pl


def kernel(frame, label, mask):
    raise NotImplementedError("write your pallas kernel here")


# speedup vs baseline: 1.7459x; 1.7459x over previous
import functools

import jax
import jax.numpy as jnp
import numpy as np
from jax import lax
from jax.experimental import pallas as pl
from jax.experimental.pallas import tpu as pltpu
from jax.experimental.pallas import tpu_sc as plsc

H, W = 64, 2048
N = 131072
HW = H * W
FOV_UP = 3.0 * np.pi / 180.0
FOV_DOWN = -25.0 * np.pi / 180.0
FOV = abs(FOV_DOWN) + abs(FOV_UP)

NT = 32
PPT = HW // NT
CHUNK = 4096
NCHUNK = N // CHUNK
ROWS_R, ROWS_C = 256, 512

INF_I32 = 0x7F800000


def _proj_body(x_ref, y_ref, z_ref, lab_ref, msk_ref, p_ref, kb_ref, lov_ref):
    x = x_ref[...]
    y = y_ref[...]
    z = z_ref[...]
    d2 = x * x + y * y + z * z
    depth = jnp.sqrt(d2) + 1e-12
    yaw = -lax.atan2(y, x)
    t = jnp.clip(z / depth, -1.0, 1.0)
    pitch = 2.0 * lax.atan2(t, 1.0 + jnp.sqrt((1.0 - t) * (1.0 + t)))
    proj_x = 0.5 * (yaw / float(np.pi) + 1.0) * W
    proj_y = (1.0 - (pitch + abs(FOV_DOWN)) / FOV) * H
    pxi = jnp.clip(jnp.floor(proj_x), 0, W - 1).astype(jnp.int32)
    pyi = jnp.clip(jnp.floor(proj_y), 0, H - 1).astype(jnp.int32)
    p = pyi * W + pxi
    p_ref[...] = ((p & 31) << 12) | (p >> 5)
    kb_ref[...] = lax.bitcast_convert_type(depth, jnp.int32)
    r = lax.broadcasted_iota(jnp.int32, x.shape, 0)
    c = lax.broadcasted_iota(jnp.int32, x.shape, 1)
    idx = r * ROWS_C + c
    lov_ref[...] = (idx << 6) | (lab_ref[...] << 1) | msk_ref[...]


_tc_proj = pl.pallas_call(
    _proj_body,
    out_shape=[
        jax.ShapeDtypeStruct((ROWS_R, ROWS_C), jnp.int32),
        jax.ShapeDtypeStruct((ROWS_R, ROWS_C), jnp.int32),
        jax.ShapeDtypeStruct((ROWS_R, ROWS_C), jnp.int32),
    ],
)


def _sc_body(p_hbm, kb_hbm, lov_hbm, fr_hbm, out5, olab, omsk,
             pb0, kb0, lb0, pb1, kb1, lb1, cpx, ckk, cll,
             hi, lo, widx, rows4, rows5, lab_v, msk_v,
             semA, semB, semG):
    cid = lax.axis_index("c")
    sid = lax.axis_index("s")
    wid = sid * 2 + cid
    base = wid * PPT

    def initb(j, carry):
        hi[pl.ds(j * 16, 16)] = jnp.full((16,), INF_I32, jnp.int32)
        lo[pl.ds(j * 16, 16)] = jnp.full((16,), 0x7FFFFFFF, jnp.int32)
        return carry

    lax.fori_loop(0, PPT // 16, initb, 0)

    def start(ci, pb, kb, lb, sem):
        off = ci * CHUNK
        pltpu.async_copy(p_hbm.at[pl.ds(off, CHUNK)], pb, sem)
        pltpu.async_copy(kb_hbm.at[pl.ds(off, CHUNK)], kb, sem)
        pltpu.async_copy(lov_hbm.at[pl.ds(off, CHUNK)], lb, sem)

    def drain(ci, pb, kb, lb, sem):
        off = ci * CHUNK
        pltpu.make_async_copy(p_hbm.at[pl.ds(off, CHUNK)], pb, sem).wait()
        pltpu.make_async_copy(kb_hbm.at[pl.ds(off, CHUNK)], kb, sem).wait()
        pltpu.make_async_copy(lov_hbm.at[pl.ds(off, CHUNK)], lb, sem).wait()

    def process(pb, kb, lb):
        def cb(v, cnt):
            sl = pl.ds(v * 16, 16)
            pp = pb[sl]
            idx = pp - base
            m = (idx >= 0) & (idx < PPT)
            dst = pl.ds(cnt, 16)
            plsc.store_compressed(cpx.at[dst], idx, mask=m)
            plsc.store_compressed(ckk.at[dst], kb[sl], mask=m)
            plsc.store_compressed(cll.at[dst], lb[sl], mask=m)
            return cnt + jnp.sum(m.astype(jnp.int32))

        cnt = lax.fori_loop(0, CHUNK // 16, cb, 0)

        def vb(v, c2):
            sl = pl.ds(v * 16, 16)
            kk = ckk[sl]
            ll = cll[sl]
            idx = cpx[sl]
            m0 = (v * 16 + lax.iota(jnp.int32, 16)) < cnt
            idxs = jnp.where(m0, idx, 0)

            def attempt(_unused):
                h = plsc.load_gather(hi, [idxs], mask=m0)
                l = plsc.load_gather(lo, [idxs], mask=m0)
                win = m0 & ((kk < h) | ((kk == h) & (ll < l)))
                plsc.store_scatter(hi, [idxs], kk, mask=win)
                plsc.store_scatter(lo, [idxs], ll, mask=win)
                return win

            attempt(None)
            win2 = attempt(None)

            @pl.when(jnp.max(win2.astype(jnp.int32)) > 0)
            def _fixup():
                lax.fori_loop(0, 14, lambda t, c3: (attempt(None), c3)[1], 0)

            return c2

        lax.fori_loop(0, (cnt + 15) // 16, vb, 0)

    start(0, pb0, kb0, lb0, semA)

    def pairb(q, carry):
        ci0 = 2 * q

        @pl.when(ci0 + 1 < NCHUNK)
        def _s1():
            start(ci0 + 1, pb1, kb1, lb1, semB)

        drain(ci0, pb0, kb0, lb0, semA)
        process(pb0, kb0, lb0)

        @pl.when(ci0 + 2 < NCHUNK)
        def _s2():
            start(ci0 + 2, pb0, kb0, lb0, semA)

        @pl.when(ci0 + 1 < NCHUNK)
        def _p1():
            drain(ci0 + 1, pb1, kb1, lb1, semB)
            process(pb1, kb1, lb1)

        return carry

    lax.fori_loop(0, (NCHUNK + 1) // 2, pairb, 0)

    def resb(j, carry):
        h = hi[pl.ds(j * 16, 16)]
        l = lo[pl.ds(j * 16, 16)]
        empty = h == INF_I32
        w = jnp.where(empty, N, l >> 6)
        widx[j // 8, pl.ds((j % 8) * 16, 16)] = w
        rowv = j * 16 + lax.iota(jnp.int32, 16)
        dvec = jnp.where(empty, 0.0, plsc.bitcast(h, jnp.float32))
        plsc.store_scatter(rows5, [rowv, jnp.full((16,), 4, jnp.int32)], dvec)
        lab_v[pl.ds(j * 16, 16)] = jnp.where(empty, -1, (l >> 1) & 31)
        msk_v[pl.ds(j * 16, 16)] = jnp.where(empty, 0, l & 1)
        return carry

    lax.fori_loop(0, PPT // 16, resb, 0)

    for g in range(2):
        handles = [
            pltpu.async_copy(fr_hbm.at[widx.at[g * 16 + r]],
                             rows4.at[pl.ds((g * 16 + r) * 128, 128)], semG)
            for r in range(16)
        ]
        for h_ in handles:
            h_.wait()

    def sb(v, carry):
        e = v * 16 + lax.iota(jnp.int32, 16)
        r4 = e >> 2
        c4 = e & 3
        vals = plsc.load_gather(rows4, [r4, c4])
        plsc.store_scatter(rows5, [r4, c4], vals)
        return carry

    lax.fori_loop(0, PPT * 4 // 16, sb, 0)

    pltpu.sync_copy(rows5, out5.at[pl.ds(base, PPT)])
    pltpu.sync_copy(lab_v, olab.at[pl.ds(base, PPT)])
    pltpu.sync_copy(msk_v, omsk.at[pl.ds(base, PPT)])


_sc_resolve = pl.kernel(
    _sc_body,
    out_type=[
        jax.ShapeDtypeStruct((HW, 5), jnp.float32),
        jax.ShapeDtypeStruct((HW,), jnp.int32),
        jax.ShapeDtypeStruct((HW,), jnp.int32),
    ],
    mesh=plsc.VectorSubcoreMesh(core_axis_name="c", subcore_axis_name="s"),
    compiler_params=pltpu.CompilerParams(
        needs_layout_passes=False, use_tc_tiling_on_sc=False),
    scratch_types=[
        pltpu.VMEM((CHUNK,), jnp.int32),
        pltpu.VMEM((CHUNK,), jnp.int32),
        pltpu.VMEM((CHUNK,), jnp.int32),
        pltpu.VMEM((CHUNK,), jnp.int32),
        pltpu.VMEM((CHUNK,), jnp.int32),
        pltpu.VMEM((CHUNK,), jnp.int32),
        pltpu.VMEM((CHUNK + 16,), jnp.int32),
        pltpu.VMEM((CHUNK + 16,), jnp.int32),
        pltpu.VMEM((CHUNK + 16,), jnp.int32),
        pltpu.VMEM((PPT,), jnp.int32),
        pltpu.VMEM((PPT,), jnp.int32),
        pltpu.VMEM((32, 128), jnp.int32),
        pltpu.VMEM((PPT, 4), jnp.float32),
        pltpu.VMEM((PPT, 5), jnp.float32),
        pltpu.VMEM((PPT,), jnp.int32),
        pltpu.VMEM((PPT,), jnp.int32),
        pltpu.SemaphoreType.DMA,
        pltpu.SemaphoreType.DMA,
        pltpu.SemaphoreType.DMA,
    ],
)


def kernel(frame, label, mask):
    xs = frame[:, 0].reshape(ROWS_R, ROWS_C)
    ys = frame[:, 1].reshape(ROWS_R, ROWS_C)
    zs = frame[:, 2].reshape(ROWS_R, ROWS_C)
    labs = label.reshape(ROWS_R, ROWS_C)
    msks = mask.astype(jnp.int32).reshape(ROWS_R, ROWS_C)
    p, kb, lov = _tc_proj(xs, ys, zs, labs, msks)
    fr_pad = jnp.concatenate(
        [frame, jnp.zeros((8, 4), jnp.float32)], axis=0)
    out5, olab, omsk = _sc_resolve(
        p.reshape(N), kb.reshape(N), lov.reshape(N), fr_pad)
    frame_img = out5.reshape(NT, HW // NT, 5).swapaxes(0, 1).reshape(H, W, 5)
    label_img = olab.reshape(NT, HW // NT).swapaxes(0, 1).reshape(H, W)
    mask_img = (omsk.reshape(NT, HW // NT).swapaxes(0, 1)
                .reshape(H, W).astype(bool))
    return frame_img, label_img, mask_img
